# baseline (speedup 1.0000x reference)
import jax
import jax.numpy as jnp
from jax import lax
from jax.experimental import pallas as pl
from jax.experimental.pallas import tpu as pltpu
from jax.experimental.pallas import tpu_sc as plsc

_N, _D = 1_000_000, 64
_Q = 249_984
_CC = 3_968
_NCH = _Q // _CC
_TAIL0 = 999_936


def _sc_body(x_hbm, o_hbm, in_buf, out_buf, in_sem, out_sem):
    wid = lax.axis_index("s") * 2 + lax.axis_index("c")
    r8 = lax.rem(wid, 8) * 8
    q = lax.div(wid, 8)
    col0 = q * _Q

    def in_copy(g, b):
        c = pl.multiple_of(col0 + g * _CC, 128)
        src = x_hbm.at[pl.ds(pl.multiple_of(r8, 8), 8), pl.ds(c, _CC)]
        return pltpu.make_async_copy(src, in_buf.at[b], in_sem.at[b])

    def out_copy(g, b):
        c = pl.multiple_of(col0 + g * _CC, 128)
        dst = o_hbm.at[pl.ds(pl.multiple_of(r8, 8), 8), pl.ds(c, _CC)]
        return pltpu.make_async_copy(out_buf.at[b], dst, out_sem.at[b])

    in_copy(0, 0).start()
    in_copy(1, 1).start()

    def chunk_body(g, _):
        b = lax.rem(g, 2)
        in_copy(g, b).wait()

        @pl.when(g >= 2)
        def _wait_prev_out():
            out_copy(g - 2, b).wait()

        def grp_body(j, _):
            for r in range(8):
                v = in_buf[b, r, pl.ds(16 * j, 16)]
                out_buf[b, r, pl.ds(16 * j, 16)] = v + 1.0
            return ()

        lax.fori_loop(0, _CC // 16, grp_body, (), unroll=2)

        @pl.when(jnp.logical_and(q == 0, g == 0))
        def _fix_col1():
            lane = lax.iota(jnp.int32, 16)
            for r in range(8):
                v = out_buf[0, r, pl.ds(0, 16)]
                out_buf[0, r, pl.ds(0, 16)] = jnp.where(
                    lane == 1, jnp.float32(2.0), v)

        out_copy(g, b).start()

        @pl.when(g + 2 < _NCH)
        def _start_next_in():
            in_copy(g + 2, b).start()

        return ()

    lax.fori_loop(0, _NCH, chunk_body, ())
    out_copy(_NCH - 2, lax.rem(_NCH - 2, 2)).wait()
    out_copy(_NCH - 1, lax.rem(_NCH - 1, 2)).wait()


@jax.jit
def _sc_add_one_t(xt):
    mesh = plsc.VectorSubcoreMesh(core_axis_name="c", subcore_axis_name="s")
    return pl.kernel(
        _sc_body,
        out_type=jax.ShapeDtypeStruct((_D, _N), jnp.float32),
        mesh=mesh,
        scratch_types=[
            pltpu.VMEM((2, 8, _CC), jnp.float32),
            pltpu.VMEM((2, 8, _CC), jnp.float32),
            pltpu.SemaphoreType.DMA((2,)),
            pltpu.SemaphoreType.DMA((2,)),
        ],
    )(xt)


def kernel(x):
    xt = x.T
    out_t = _sc_add_one_t(xt)
    tail = lax.slice(xt, (0, _TAIL0), (_D, _N))
    out_t = lax.dynamic_update_slice(out_t, tail + 1.0, (0, _TAIL0))
    return out_t.T


# speedup vs baseline: 3.6213x; 3.6213x over previous
import jax
import jax.numpy as jnp
from jax.experimental import pallas as pl
from jax.experimental.pallas import tpu as pltpu

_N, _D = 1_000_000, 64
_BC = 40_960


def _body(x_ref, o_ref):
    o_ref[...] = x_ref[...] + 1.0

    @pl.when(pl.program_id(0) == 0)
    def _fix_col1():
        o_ref[:, 1] = jnp.full((_D,), 2.0, dtype=o_ref.dtype)


def kernel(x):
    xt = x.T
    grid = (_N + _BC - 1) // _BC
    out_t = pl.pallas_call(
        _body,
        grid=(grid,),
        in_specs=[pl.BlockSpec((_D, _BC), lambda j: (0, j))],
        out_specs=pl.BlockSpec((_D, _BC), lambda j: (0, j)),
        out_shape=jax.ShapeDtypeStruct((_D, _N), jnp.float32),
    )(xt)
    return out_t.T
